# baseline (speedup 1.0000x reference)
import dataclasses
import functools

import jax
import jax.numpy as jnp
from jax import lax
from jax.experimental import pallas as pl
from jax.experimental.pallas import tpu as pltpu
from jax.experimental.pallas import tpu_sc as plsc

_NUM_CORES = 2
_NUM_SUBCORES = 16
_NW = _NUM_CORES * _NUM_SUBCORES


@functools.partial(jax.jit, static_argnames=("bb", "t"))
def _sc_lookup(table, idx, bb, t):
    V, D = table.shape
    (B,) = idx.shape
    assert B == bb * t and B % (8 * _NW) == 0
    b_per_w = B // _NW

    mesh = plsc.VectorSubcoreMesh(core_axis_name="c", subcore_axis_name="s")
    cp = pltpu.CompilerParams()
    if "use_tc_tiling_on_sc" in pltpu.CompilerParams.__dataclass_fields__:
        cp = dataclasses.replace(cp, use_tc_tiling_on_sc=True)

    @functools.partial(
        pl.kernel,
        mesh=mesh,
        out_type=jax.ShapeDtypeStruct((bb, t, D), jnp.float32),
        compiler_params=cp,
        scratch_types=[
            pltpu.VMEM((V, D), jnp.float32),
            pltpu.VMEM((b_per_w,), jnp.int32),
            pltpu.SemaphoreType.DMA,
        ],
    )
    def k(table_hbm, idx_hbm, out_hbm, table_v, idx_v, wsem):
        wid = lax.axis_index("s") * _NUM_CORES + lax.axis_index("c")
        base = wid * b_per_w
        pltpu.sync_copy(table_hbm, table_v)
        pltpu.sync_copy(idx_hbm.at[pl.ds(base, b_per_w)], idx_v)

        def wait_row():
            pltpu.make_async_copy(table_v.at[0], out_hbm.at[0, 0], wsem).wait()

        n_groups = b_per_w // 16

        @pl.loop(0, n_groups)
        def _(g):
            vec = idx_v[pl.ds(g * 16, 16)]
            j0 = base + g * 16
            for l in range(16):
                j = j0 + l
                pltpu.async_copy(
                    table_v.at[vec[l]], out_hbm.at[j // t, j % t], wsem
                )
            for _ in range(16):
                wait_row()

    return k(table, idx)


def kernel(indices, embedding_weight):
    bb, t = indices.shape
    flat_idx = indices.reshape(-1).astype(jnp.int32)
    return _sc_lookup(embedding_weight, flat_idx, bb, t)


# speedup vs baseline: 1.0127x; 1.0127x over previous
import dataclasses
import functools

import jax
import jax.numpy as jnp
from jax import lax
from jax.experimental import pallas as pl
from jax.experimental.pallas import tpu as pltpu
from jax.experimental.pallas import tpu_sc as plsc

_NUM_CORES = 2
_NUM_SUBCORES = 16
_NW = _NUM_CORES * _NUM_SUBCORES


@functools.partial(jax.jit, static_argnames=("bb", "t"))
def _sc_lookup(table, idx, bb, t):
    V, D = table.shape
    (B,) = idx.shape
    assert B == bb * t and B % (8 * _NW) == 0
    b_per_w = B // _NW

    mesh = plsc.VectorSubcoreMesh(core_axis_name="c", subcore_axis_name="s")
    cp = pltpu.CompilerParams()
    if "use_tc_tiling_on_sc" in pltpu.CompilerParams.__dataclass_fields__:
        cp = dataclasses.replace(cp, use_tc_tiling_on_sc=True)

    @functools.partial(
        pl.kernel,
        mesh=mesh,
        out_type=jax.ShapeDtypeStruct((bb, t, D), jnp.float32),
        compiler_params=cp,
        scratch_types=[
            pltpu.VMEM((V, D), jnp.float32),
            pltpu.VMEM((b_per_w,), jnp.int32),
            pltpu.SemaphoreType.DMA,
        ],
    )
    def k(table_hbm, idx_hbm, out_hbm, table_v, idx_v, wsem):
        wid = lax.axis_index("s") * _NUM_CORES + lax.axis_index("c")
        base = wid * b_per_w
        pltpu.sync_copy(table_hbm, table_v)
        pltpu.sync_copy(idx_hbm.at[pl.ds(base, b_per_w)], idx_v)

        def wait_row():
            pltpu.make_async_copy(table_v.at[0], out_hbm.at[0, 0], wsem).wait()

        n_groups = b_per_w // 16

        def fire_group(g):
            vec = idx_v[pl.ds(g * 16, 16)]
            j0 = base + g * 16
            for l in range(16):
                j = j0 + l
                pltpu.async_copy(
                    table_v.at[vec[l]], out_hbm.at[j // t, j % t], wsem
                )

        fire_group(0)

        @pl.loop(1, n_groups)
        def _(g):
            fire_group(g)
            for _ in range(16):
                wait_row()

        for _ in range(16):
            wait_row()

    return k(table, idx)


def kernel(indices, embedding_weight):
    bb, t = indices.shape
    flat_idx = indices.reshape(-1).astype(jnp.int32)
    return _sc_lookup(embedding_weight, flat_idx, bb, t)
